# baseline (speedup 1.0000x reference)
import dataclasses
import functools

import jax
import jax.numpy as jnp
from jax import lax
from jax.experimental import pallas as pl
from jax.experimental.pallas import tpu as pltpu
from jax.experimental.pallas import tpu_sc as plsc

_N = 10000
_IN_C = 128
_OUT_C = 64
_HID = 128
_E = 320000

_NC = 2
_NS = 16
_NW = _NC * _NS
_CHUNK = 128
_K0 = 104
_K1 = 56
_KMAX = max(_K0, _K1)
_NCHUNK = _NS * (_K0 + _K1)
_CPW = _NCHUNK // _NW
_E_PAD = _NW * _CPW * _CHUNK
_N_PAD = 10112
_RPT = _N_PAD // _NS

_ROWS_BLK = 1000

_mesh = plsc.VectorSubcoreMesh(
    core_axis_name="c", subcore_axis_name="s", num_cores=_NC, num_subcores=_NS
)

_cp = pltpu.CompilerParams()
if "needs_layout_passes" in pltpu.CompilerParams.__dataclass_fields__:
    _cp = dataclasses.replace(_cp, needs_layout_passes=False)


def _sc_count(dst3):
    @functools.partial(
        pl.kernel,
        out_type=jax.ShapeDtypeStruct((_NW, _N_PAD), jnp.float32),
        mesh=_mesh,
        compiler_params=_cp,
        scratch_types=[
            pltpu.VMEM((_CPW, _CHUNK), jnp.int32),
            pltpu.VMEM((_N_PAD,), jnp.float32),
        ],
    )
    def k(dst_hbm, out_hbm, dstv, acc):
        c = lax.axis_index("c")
        s = lax.axis_index("s")
        wid = c * _NS + s
        pltpu.sync_copy(dst_hbm.at[pl.ds(wid * _CPW, _CPW)], dstv)
        zeros16 = jnp.zeros((16,), jnp.float32)

        @pl.loop(0, _N_PAD // 16)
        def _(i):
            acc[pl.ds(i * 16, 16)] = zeros16

        ones16 = jnp.ones((16,), jnp.float32)

        @pl.loop(0, _CPW)
        def _(i):
            for j in range(_CHUNK // 16):
                idx = dstv[i, pl.ds(j * 16, 16)]
                plsc.addupdate_scatter(acc, [idx], ones16)

        pltpu.sync_copy(acc, out_hbm.at[wid])

    return k(dst3)


def _sc_prop(g, src3, dst3, zeros_feat):
    @functools.partial(
        pl.kernel,
        out_type=jax.ShapeDtypeStruct((_NC, _N_PAD, _HID), jnp.float32),
        mesh=_mesh,
        scratch_types=[
            pltpu.VMEM((_KMAX, _CHUNK), jnp.int32),
            pltpu.VMEM((_KMAX, _CHUNK), jnp.int32),
            pltpu.VMEM((_CHUNK, _HID), jnp.float32),
            pltpu.VMEM_SHARED((_N_PAD, _HID), jnp.float32),
            pltpu.SemaphoreType.DMA,
        ],
    )
    def k(g_hbm, src_hbm, dst_hbm, zeros_hbm, out_hbm, srcv, dstv, rows, acc,
          sem):
        c = lax.axis_index("c")
        s = lax.axis_index("s")
        pltpu.sync_copy(
            zeros_hbm.at[pl.ds(s * _RPT, _RPT)], acc.at[pl.ds(s * _RPT, _RPT)]
        )

        def run(base, k):
            pltpu.sync_copy(src_hbm.at[pl.ds(base, k)], srcv.at[pl.ds(0, k)])
            pltpu.sync_copy(dst_hbm.at[pl.ds(base, k)], dstv.at[pl.ds(0, k)])
            plsc.subcore_barrier()

            @pl.loop(0, k)
            def _(i):
                pltpu.async_copy(g_hbm.at[srcv.at[i]], rows, sem).wait()
                pltpu.sync_copy(rows, acc.at[dstv.at[i]], add=True)

        @pl.when(c == 0)
        def _():
            run(s * _K0, _K0)

        @pl.when(c == 1)
        def _():
            run(_NS * _K0 + s * _K1, _K1)

        plsc.subcore_barrier()
        pltpu.sync_copy(
            acc.at[pl.ds(s * _RPT, _RPT)], out_hbm.at[c, pl.ds(s * _RPT, _RPT)]
        )

    return k(g, src3, dst3, zeros_feat)


def _tc_matmul(x, w):
    def body(x_ref, w_ref, o_ref):
        o_ref[...] = jnp.dot(
            x_ref[...], w_ref[...], preferred_element_type=jnp.float32
        )

    return pl.pallas_call(
        body,
        grid=(_N // _ROWS_BLK,),
        in_specs=[
            pl.BlockSpec((_ROWS_BLK, _IN_C), lambda i: (i, 0)),
            pl.BlockSpec((_IN_C, _HID), lambda i: (0, 0)),
        ],
        out_specs=pl.BlockSpec((_ROWS_BLK, _HID), lambda i: (i, 0)),
        out_shape=jax.ShapeDtypeStruct((_N, _HID), jnp.float32),
    )(x, w)


def _tc_dis(cnt):
    def body(c_ref, o_ref):
        o_ref[...] = lax.rsqrt(jnp.sum(c_ref[...], axis=0) + 1.0)[:, None]

    return pl.pallas_call(
        body,
        out_shape=jax.ShapeDtypeStruct((_N_PAD, 1), jnp.float32),
    )(cnt)


def _tc_scale(dis, t):
    def body(d_ref, t_ref, o_ref):
        o_ref[...] = d_ref[...] * t_ref[...]

    return pl.pallas_call(
        body,
        grid=(_N // _ROWS_BLK,),
        in_specs=[
            pl.BlockSpec((_ROWS_BLK, 1), lambda i: (i, 0)),
            pl.BlockSpec((_ROWS_BLK, _HID), lambda i: (i, 0)),
        ],
        out_specs=pl.BlockSpec((_ROWS_BLK, _HID), lambda i: (i, 0)),
        out_shape=jax.ShapeDtypeStruct((_N, _HID), jnp.float32),
    )(dis, t)


def _tc_bn(p, g1, dis, b1, gamma, beta):
    def body(p_ref, g_ref, d_ref, b_ref, ga_ref, be_ref, o_ref):
        d = d_ref[:_N, 0]
        h = d[:, None] * (p_ref[0, :_N, :] + p_ref[1, :_N, :] + g_ref[...])
        h = h + b_ref[0, :]
        mean = jnp.mean(h, axis=0)
        var = jnp.mean((h - mean) ** 2, axis=0)
        hn = (h - mean) * lax.rsqrt(var + 1e-5) * ga_ref[0, :] + be_ref[0, :]
        o_ref[...] = d[:, None] * jnp.maximum(hn, 0.0)

    return pl.pallas_call(
        body,
        out_shape=jax.ShapeDtypeStruct((_N, _HID), jnp.float32),
    )(p, g1, dis, b1, gamma, beta)


def _tc_out(p, g2, dis, wml, bml):
    def body(p_ref, g_ref, d_ref, w_ref, b_ref, o_ref):
        q = d_ref[...] * (p_ref[0] + p_ref[1] + g_ref[...])
        o_ref[...] = (
            jnp.dot(q, w_ref[...], preferred_element_type=jnp.float32) + b_ref[0, :]
        )

    return pl.pallas_call(
        body,
        grid=(_N // _ROWS_BLK,),
        in_specs=[
            pl.BlockSpec((2, _ROWS_BLK, _HID), lambda i: (0, i, 0)),
            pl.BlockSpec((_ROWS_BLK, _HID), lambda i: (i, 0)),
            pl.BlockSpec((_ROWS_BLK, 1), lambda i: (i, 0)),
            pl.BlockSpec((_HID, _HID), lambda i: (0, 0)),
            pl.BlockSpec((1, _HID), lambda i: (0, 0)),
        ],
        out_specs=pl.BlockSpec((_ROWS_BLK, _HID), lambda i: (i, 0)),
        out_shape=jax.ShapeDtypeStruct((_N, _HID), jnp.float32),
    )(p, g2, dis, wml, bml)


def kernel(x, edge_index, W1, b1, gamma, beta, Wmu, bmu, Wls, bls):
    src = edge_index[0]
    dst = edge_index[1]
    pad = _NCHUNK * _CHUNK - _E
    src2 = jnp.concatenate([src, jnp.zeros((pad,), jnp.int32)]).reshape(
        _NCHUNK, _CHUNK
    )
    dst2 = jnp.concatenate([dst, jnp.full((pad,), _N, jnp.int32)]).reshape(
        _NCHUNK, _CHUNK
    )
    zeros_feat = jnp.zeros((_N_PAD, _HID), jnp.float32)

    cnt = _sc_count(dst2)
    t1 = _tc_matmul(x, W1)
    dis = _tc_dis(cnt)
    g1 = _tc_scale(dis, t1)
    p1 = _sc_prop(g1, src2, dst2, zeros_feat)
    g2 = _tc_bn(
        p1, g1, dis, b1.reshape(1, -1), gamma.reshape(1, -1), beta.reshape(1, -1)
    )
    p2 = _sc_prop(g2, src2, dst2, zeros_feat)
    wml = jnp.concatenate([Wmu, Wls], axis=1)
    bml = jnp.concatenate([bmu, bls]).reshape(1, -1)
    y = _tc_out(p2, g2, dis, wml, bml)
    return y[:, :_OUT_C], y[:, _OUT_C:]


# speedup vs baseline: 1.6346x; 1.6346x over previous
import dataclasses
import functools

import jax
import jax.numpy as jnp
from jax import lax
from jax.experimental import pallas as pl
from jax.experimental.pallas import tpu as pltpu
from jax.experimental.pallas import tpu_sc as plsc

_N = 10000
_IN_C = 128
_OUT_C = 64
_HID = 128
_E = 320000

_NC = 2
_NS = 16
_NW = _NC * _NS
_CHUNK = 128
_CPW = 79
_E_PAD = _NW * _CPW * _CHUNK
_N_PAD = 10112
_RPT = _N_PAD // _NS

_ROWS_BLK = 1000

_mesh = plsc.VectorSubcoreMesh(
    core_axis_name="c", subcore_axis_name="s", num_cores=_NC, num_subcores=_NS
)

_cp = pltpu.CompilerParams()
if "needs_layout_passes" in pltpu.CompilerParams.__dataclass_fields__:
    _cp = dataclasses.replace(_cp, needs_layout_passes=False)


def _sc_count(dst3):
    @functools.partial(
        pl.kernel,
        out_type=jax.ShapeDtypeStruct((_NW, _N_PAD), jnp.float32),
        mesh=_mesh,
        compiler_params=_cp,
        scratch_types=[
            pltpu.VMEM((_CPW, _CHUNK), jnp.int32),
            pltpu.VMEM((_N_PAD,), jnp.float32),
        ],
    )
    def k(dst_hbm, out_hbm, dstv, acc):
        c = lax.axis_index("c")
        s = lax.axis_index("s")
        wid = c * _NS + s
        pltpu.sync_copy(dst_hbm.at[wid], dstv)
        zeros16 = jnp.zeros((16,), jnp.float32)

        @pl.loop(0, _N_PAD // 16)
        def _(i):
            acc[pl.ds(i * 16, 16)] = zeros16

        ones16 = jnp.ones((16,), jnp.float32)

        @pl.loop(0, _CPW)
        def _(i):
            for j in range(_CHUNK // 16):
                idx = dstv[i, pl.ds(j * 16, 16)]
                plsc.addupdate_scatter(acc, [idx], ones16)

        pltpu.sync_copy(acc, out_hbm.at[wid])

    return k(dst3)


def _sc_prop(g, src3, dst3, zeros_feat):
    @functools.partial(
        pl.kernel,
        out_type=jax.ShapeDtypeStruct((_NC, _N_PAD, _HID), jnp.float32),
        mesh=_mesh,
        scratch_types=[
            pltpu.VMEM((_CPW, _CHUNK), jnp.int32),
            pltpu.VMEM((_CPW, _CHUNK), jnp.int32),
            pltpu.VMEM((_CHUNK, _HID), jnp.float32),
            pltpu.VMEM_SHARED((_N_PAD, _HID), jnp.float32),
            pltpu.SemaphoreType.DMA,
        ],
    )
    def k(g_hbm, src_hbm, dst_hbm, zeros_hbm, out_hbm, srcv, dstv, rows, acc,
          sem):
        c = lax.axis_index("c")
        s = lax.axis_index("s")
        wid = c * _NS + s
        pltpu.sync_copy(src_hbm.at[wid], srcv)
        pltpu.sync_copy(dst_hbm.at[wid], dstv)
        pltpu.sync_copy(
            zeros_hbm.at[pl.ds(s * _RPT, _RPT)], acc.at[pl.ds(s * _RPT, _RPT)]
        )
        plsc.subcore_barrier()

        @pl.loop(0, _CPW)
        def _(i):
            pltpu.async_copy(g_hbm.at[srcv.at[i]], rows, sem).wait()
            pltpu.sync_copy(rows, acc.at[dstv.at[i]], add=True)

        plsc.subcore_barrier()
        pltpu.sync_copy(
            acc.at[pl.ds(s * _RPT, _RPT)], out_hbm.at[c, pl.ds(s * _RPT, _RPT)]
        )

    return k(g, src3, dst3, zeros_feat)


def _tc_matmul(x, w):
    def body(x_ref, w_ref, o_ref):
        o_ref[...] = jnp.dot(
            x_ref[...], w_ref[...], preferred_element_type=jnp.float32
        )

    return pl.pallas_call(
        body,
        grid=(_N // _ROWS_BLK,),
        in_specs=[
            pl.BlockSpec((_ROWS_BLK, _IN_C), lambda i: (i, 0)),
            pl.BlockSpec((_IN_C, _HID), lambda i: (0, 0)),
        ],
        out_specs=pl.BlockSpec((_ROWS_BLK, _HID), lambda i: (i, 0)),
        out_shape=jax.ShapeDtypeStruct((_N, _HID), jnp.float32),
    )(x, w)


def _tc_dis(cnt):
    def body(c_ref, o_ref):
        o_ref[...] = lax.rsqrt(jnp.sum(c_ref[...], axis=0) + 1.0)[:, None]

    return pl.pallas_call(
        body,
        out_shape=jax.ShapeDtypeStruct((_N_PAD, 1), jnp.float32),
    )(cnt)


def _tc_scale(dis, t):
    def body(d_ref, t_ref, o_ref):
        o_ref[...] = d_ref[...] * t_ref[...]

    return pl.pallas_call(
        body,
        grid=(_N // _ROWS_BLK,),
        in_specs=[
            pl.BlockSpec((_ROWS_BLK, 1), lambda i: (i, 0)),
            pl.BlockSpec((_ROWS_BLK, _HID), lambda i: (i, 0)),
        ],
        out_specs=pl.BlockSpec((_ROWS_BLK, _HID), lambda i: (i, 0)),
        out_shape=jax.ShapeDtypeStruct((_N, _HID), jnp.float32),
    )(dis, t)


def _tc_bn(p, g1, dis, b1, gamma, beta):
    def body(p_ref, g_ref, d_ref, b_ref, ga_ref, be_ref, o_ref):
        d = d_ref[:_N, 0]
        h = d[:, None] * (p_ref[0, :_N, :] + p_ref[1, :_N, :] + g_ref[...])
        h = h + b_ref[0, :]
        mean = jnp.mean(h, axis=0)
        var = jnp.mean((h - mean) ** 2, axis=0)
        hn = (h - mean) * lax.rsqrt(var + 1e-5) * ga_ref[0, :] + be_ref[0, :]
        o_ref[...] = d[:, None] * jnp.maximum(hn, 0.0)

    return pl.pallas_call(
        body,
        out_shape=jax.ShapeDtypeStruct((_N, _HID), jnp.float32),
    )(p, g1, dis, b1, gamma, beta)


def _tc_out(p, g2, dis, wml, bml):
    def body(p_ref, g_ref, d_ref, w_ref, b_ref, o_ref):
        q = d_ref[...] * (p_ref[0] + p_ref[1] + g_ref[...])
        o_ref[...] = (
            jnp.dot(q, w_ref[...], preferred_element_type=jnp.float32) + b_ref[0, :]
        )

    return pl.pallas_call(
        body,
        grid=(_N // _ROWS_BLK,),
        in_specs=[
            pl.BlockSpec((2, _ROWS_BLK, _HID), lambda i: (0, i, 0)),
            pl.BlockSpec((_ROWS_BLK, _HID), lambda i: (i, 0)),
            pl.BlockSpec((_ROWS_BLK, 1), lambda i: (i, 0)),
            pl.BlockSpec((_HID, _HID), lambda i: (0, 0)),
            pl.BlockSpec((1, _HID), lambda i: (0, 0)),
        ],
        out_specs=pl.BlockSpec((_ROWS_BLK, _HID), lambda i: (i, 0)),
        out_shape=jax.ShapeDtypeStruct((_N, _HID), jnp.float32),
    )(p, g2, dis, wml, bml)


def kernel(x, edge_index, W1, b1, gamma, beta, Wmu, bmu, Wls, bls):
    src = edge_index[0]
    dst = edge_index[1]
    pad = _E_PAD - _E
    src3 = jnp.concatenate([src, jnp.zeros((pad,), jnp.int32)]).reshape(
        _NW, _CPW, _CHUNK
    )
    dst3 = jnp.concatenate([dst, jnp.full((pad,), _N, jnp.int32)]).reshape(
        _NW, _CPW, _CHUNK
    )
    zeros_feat = jnp.zeros((_N_PAD, _HID), jnp.float32)

    cnt = _sc_count(dst3)
    t1 = _tc_matmul(x, W1)
    dis = _tc_dis(cnt)
    g1 = _tc_scale(dis, t1)
    p1 = _sc_prop(g1, src3, dst3, zeros_feat)
    g2 = _tc_bn(
        p1, g1, dis, b1.reshape(1, -1), gamma.reshape(1, -1), beta.reshape(1, -1)
    )
    p2 = _sc_prop(g2, src3, dst3, zeros_feat)
    wml = jnp.concatenate([Wmu, Wls], axis=1)
    bml = jnp.concatenate([bmu, bls]).reshape(1, -1)
    y = _tc_out(p2, g2, dis, wml, bml)
    return y[:, :_OUT_C], y[:, _OUT_C:]
